# baseline (speedup 1.0000x reference)
import functools

import jax
import jax.numpy as jnp
from jax import lax
from jax.experimental import pallas as pl
from jax.experimental.pallas import tpu as pltpu
from jax.experimental.pallas import tpu_sc as plsc

N = 10000
F = 128
H = 128
O = 64
E = 320000

NC = 2
NS = 16
NW = NC * NS
EPW = E // NW
CH = 80
NCHUNK = EPW // CH
RPW = 624

BLK = 400


def _spmm_body(src_hbm, dst_hbm, ew_hbm, x_hbm, zero_hbm, out_hbm,
               src_all, dst_v0, dst_v1, dst_v2, ew_v0, ew_v1, ew_v2,
               rows_v0, rows_v1, rows_v2, acc_sh,
               g0, g1, g2, s0, s1, s2):
    c = lax.axis_index("c")
    s = lax.axis_index("s")
    wid = s * NC + c
    base = wid * EPW
    rows = (rows_v0, rows_v1, rows_v2)
    dstv = (dst_v0, dst_v1, dst_v2)
    ewv = (ew_v0, ew_v1, ew_v2)
    gsem = (g0, g1, g2)
    ssem = (s0, s1, s2)

    @pl.when(s == 0)
    def _():
        pltpu.sync_copy(zero_hbm, acc_sh)

    pltpu.sync_copy(src_hbm.at[pl.ds(base, EPW)], src_all)
    plsc.subcore_barrier()

    def issue(t, j):
        off = t * CH
        pltpu.async_copy(x_hbm.at[src_all.at[pl.ds(off, CH)]], rows[j],
                         gsem[j])
        pltpu.async_copy(dst_hbm.at[pl.ds(base + off, CH)], dstv[j], gsem[j])
        pltpu.async_copy(ew_hbm.at[pl.ds(base + off, CH)], ewv[j], gsem[j])

    def wait_gather(j):
        pltpu.make_async_copy(x_hbm.at[src_all.at[pl.ds(0, CH)]], rows[j],
                              gsem[j]).wait()
        pltpu.make_async_copy(dst_hbm.at[pl.ds(0, CH)], dstv[j],
                              gsem[j]).wait()
        pltpu.make_async_copy(ew_hbm.at[pl.ds(0, CH)], ewv[j],
                              gsem[j]).wait()

    def wait_scatter(j):
        pltpu.make_async_copy(rows[j], acc_sh.at[dstv[j]], ssem[j]).wait()

    def section(t, j, refill_wait, refill):
        wait_gather(j)
        for g in range(CH // 16):
            ew16 = ewv[j][pl.ds(g * 16, 16)]
            for q in range(16):
                e = g * 16 + q
                w = jnp.full((16,), ew16[q], jnp.float32)
                for k in range(H // 16):
                    rows[j][e, pl.ds(16 * k, 16)] = (
                        rows[j][e, pl.ds(16 * k, 16)] * w)
        pltpu.async_copy(rows[j], acc_sh.at[dstv[j]], ssem[j], add=True)
        jm1 = (j - 1) % 3
        if refill_wait:
            wait_scatter(jm1)
        if refill:
            issue(t + 2, jm1)

    issue(0, 0)
    issue(1, 1)
    section(0, 0, False, True)
    section(1, 1, True, True)
    section(2, 2, True, True)

    def body(i, carry):
        t = 3 * i
        section(t, 0, True, True)
        section(t + 1, 1, True, True)
        section(t + 2, 2, True, True)
        return carry

    lax.fori_loop(1, (NCHUNK - 2) // 3, body, 0)
    section(NCHUNK - 2, 0, False, False)
    section(NCHUNK - 1, 1, False, False)
    wait_scatter(0)
    wait_scatter(1)
    wait_scatter(2)
    plsc.subcore_barrier()
    pltpu.sync_copy(acc_sh.at[pl.ds(s * RPW, RPW)],
                    out_hbm.at[c, pl.ds(s * RPW, RPW)])

    @pl.when(s == NS - 1)
    def _():
        pltpu.sync_copy(acc_sh.at[pl.ds(NS * RPW, N - NS * RPW)],
                        out_hbm.at[c, pl.ds(NS * RPW, N - NS * RPW)])


_spmm_sc = pl.kernel(
    _spmm_body,
    out_type=jax.ShapeDtypeStruct((NC, N, H), jnp.float32),
    mesh=plsc.VectorSubcoreMesh(core_axis_name="c", subcore_axis_name="s"),
    scratch_types=(
        [pltpu.VMEM((EPW,), jnp.int32)]
        + [pltpu.VMEM((CH,), jnp.int32)] * 3
        + [pltpu.VMEM((CH,), jnp.float32)] * 3
        + [pltpu.VMEM((CH, H), jnp.float32)] * 3
        + [pltpu.VMEM_SHARED((N, H), jnp.float32)]
        + [pltpu.SemaphoreType.DMA] * 6
    ),
)


def _relu_body(gp_ref, b_ref, out_ref):
    out_ref[...] = jnp.maximum(gp_ref[0] + gp_ref[1] + b_ref[...], 0.0)


def _hidden_g(g1p, gcn_hidden_bias):
    return pl.pallas_call(
        _relu_body,
        grid=(N // BLK,),
        in_specs=[
            pl.BlockSpec((NC, BLK, H), lambda i: (0, i, 0)),
            pl.BlockSpec((H,), lambda i: (0,)),
        ],
        out_specs=pl.BlockSpec((BLK, H), lambda i: (i, 0)),
        out_shape=jax.ShapeDtypeStruct((N, H), jnp.float32),
    )(g1p, gcn_hidden_bias)


def _dense_body(x_ref, hw_ref, hb_ref, mw_ref, mb_ref, lw_ref, lb_ref,
                g2p_ref, mix_ref, zm_ref, zs_ref):
    hm = jnp.maximum(x_ref[...] @ hw_ref[...] + hb_ref[...], 0.0)
    zm_mlp = hm @ mw_ref[...] + mb_ref[...]
    zs_mlp = hm @ lw_ref[...] + lb_ref[...]
    g2 = g2p_ref[0] + g2p_ref[1]
    zm_gcn = g2 @ mw_ref[...]
    zs_gcn = g2 @ lw_ref[...]
    w = mix_ref[0, 0]
    r = jax.nn.sigmoid(w)
    zm_ref[...] = zm_gcn * w + zm_mlp * (1.0 - w)
    zs_ref[...] = zs_gcn * r + zs_mlp * (1.0 - r)


def _dense_fuse(x, g2p, mixture_weight, hidden_weight, hidden_bias,
                mean_weight, mean_bias, log_std_weight, log_std_bias):
    mix = mixture_weight.reshape(1, 1)
    row = pl.BlockSpec((BLK, H), lambda i: (i, 0))
    full = lambda shape: pl.BlockSpec(shape, lambda i: tuple(0 for _ in shape))
    return pl.pallas_call(
        _dense_body,
        grid=(N // BLK,),
        in_specs=[
            row,
            full((F, H)), full((H,)),
            full((H, O)), full((O,)),
            full((H, O)), full((O,)),
            pl.BlockSpec((NC, BLK, H), lambda i: (0, i, 0)),
            full((1, 1)),
        ],
        out_specs=[pl.BlockSpec((BLK, O), lambda i: (i, 0))] * 2,
        out_shape=[jax.ShapeDtypeStruct((N, O), jnp.float32)] * 2,
    )(x, hidden_weight, hidden_bias, mean_weight, mean_bias,
      log_std_weight, log_std_bias, g2p, mix)


def kernel(input, edge_index, edge_weight, mixture_weight, hidden_weight,
           hidden_bias, gcn_hidden_weight, gcn_hidden_bias, mean_weight,
           mean_bias, log_std_weight, log_std_bias):
    dst = edge_index[0]
    src = edge_index[1]
    zeros = jnp.zeros((N, H), jnp.float32)
    g1p = _spmm_sc(src, dst, edge_weight, gcn_hidden_weight, zeros)
    hidden_g = _hidden_g(g1p, gcn_hidden_bias)
    g2p = _spmm_sc(src, dst, edge_weight, hidden_g, zeros)
    zm, zs = _dense_fuse(input, g2p, mixture_weight, hidden_weight,
                         hidden_bias, mean_weight, mean_bias,
                         log_std_weight, log_std_bias)
    return (zm, zs)


# speedup vs baseline: 1.1176x; 1.1176x over previous
import functools

import jax
import jax.numpy as jnp
from jax import lax
from jax.experimental import pallas as pl
from jax.experimental.pallas import tpu as pltpu
from jax.experimental.pallas import tpu_sc as plsc

N = 10000
F = 128
H = 128
O = 64
E = 320000

NC = 2
NS = 16
NW = NC * NS
EPW = E // NW
CH = 80
NCHUNK = EPW // CH
RPW = 624

BLK = 400


def _spmm_body(src_hbm, dst_hbm, ew_hbm, x_hbm, zero_hbm, out_hbm,
               src_all, ew_all, dst_all, rows_v0, rows_v1,
               acc_sh, sem0, sem1):
    c = lax.axis_index("c")
    s = lax.axis_index("s")
    wid = s * NC + c
    base = wid * EPW
    rows = (rows_v0, rows_v1)
    sems = (sem0, sem1)

    @pl.when(s == 0)
    def _():
        pltpu.sync_copy(zero_hbm, acc_sh)

    pltpu.sync_copy(src_hbm.at[pl.ds(base, EPW)], src_all)
    pltpu.sync_copy(ew_hbm.at[pl.ds(base, EPW)], ew_all)
    pltpu.sync_copy(dst_hbm.at[pl.ds(base, EPW)], dst_all)
    plsc.subcore_barrier()

    def issue(t, p):
        off = t * CH
        pltpu.async_copy(x_hbm.at[src_all.at[pl.ds(off, CH)]], rows[p],
                         sems[p])

    def wait(p):
        pltpu.make_async_copy(x_hbm.at[src_all.at[pl.ds(0, CH)]], rows[p],
                              sems[p]).wait()

    def process(t, p, guard_next):
        wait(p)
        off = t * CH
        for g in range(CH // 16):
            ew16 = ew_all[pl.ds(off + g * 16, 16)]
            for j in range(16):
                e = g * 16 + j
                w = jnp.full((16,), ew16[j], jnp.float32)
                for k in range(H // 16):
                    rows[p][e, pl.ds(16 * k, 16)] = (
                        rows[p][e, pl.ds(16 * k, 16)] * w)
        pltpu.sync_copy(rows[p], acc_sh.at[dst_all.at[pl.ds(off, CH)]],
                        add=True)
        if guard_next:
            @pl.when(t + 2 < NCHUNK)
            def _():
                issue(t + 2, p)
        else:
            issue(t + 2, p)

    issue(0, 0)
    issue(1, 1)

    def pair(i, carry):
        t = i * 2
        process(t, 0, False)
        process(t + 1, 1, True)
        return carry

    lax.fori_loop(0, (NCHUNK - 1) // 2, pair, 0)
    process(NCHUNK - 1, 0, True)
    plsc.subcore_barrier()
    pltpu.sync_copy(acc_sh.at[pl.ds(s * RPW, RPW)],
                    out_hbm.at[c, pl.ds(s * RPW, RPW)])

    @pl.when(s == NS - 1)
    def _():
        pltpu.sync_copy(acc_sh.at[pl.ds(NS * RPW, N - NS * RPW)],
                        out_hbm.at[c, pl.ds(NS * RPW, N - NS * RPW)])


_spmm_sc = pl.kernel(
    _spmm_body,
    out_type=jax.ShapeDtypeStruct((NC, N, H), jnp.float32),
    mesh=plsc.VectorSubcoreMesh(core_axis_name="c", subcore_axis_name="s"),
    scratch_types=[
        pltpu.VMEM((EPW,), jnp.int32),
        pltpu.VMEM((EPW,), jnp.float32),
        pltpu.VMEM((EPW,), jnp.int32),
        pltpu.VMEM((CH, H), jnp.float32),
        pltpu.VMEM((CH, H), jnp.float32),
        pltpu.VMEM_SHARED((N, H), jnp.float32),
        pltpu.SemaphoreType.DMA,
        pltpu.SemaphoreType.DMA,
    ],
)


def _relu_body(gp_ref, b_ref, out_ref):
    out_ref[...] = jnp.maximum(gp_ref[0] + gp_ref[1] + b_ref[...], 0.0)


def _hidden_g(g1p, gcn_hidden_bias):
    return pl.pallas_call(
        _relu_body,
        grid=(N // BLK,),
        in_specs=[
            pl.BlockSpec((NC, BLK, H), lambda i: (0, i, 0)),
            pl.BlockSpec((H,), lambda i: (0,)),
        ],
        out_specs=pl.BlockSpec((BLK, H), lambda i: (i, 0)),
        out_shape=jax.ShapeDtypeStruct((N, H), jnp.float32),
    )(g1p, gcn_hidden_bias)


def _dense_body(x_ref, hw_ref, hb_ref, mw_ref, mb_ref, lw_ref, lb_ref,
                g2p_ref, mix_ref, zm_ref, zs_ref):
    hm = jnp.maximum(x_ref[...] @ hw_ref[...] + hb_ref[...], 0.0)
    zm_mlp = hm @ mw_ref[...] + mb_ref[...]
    zs_mlp = hm @ lw_ref[...] + lb_ref[...]
    g2 = g2p_ref[0] + g2p_ref[1]
    zm_gcn = g2 @ mw_ref[...]
    zs_gcn = g2 @ lw_ref[...]
    w = mix_ref[0, 0]
    r = jax.nn.sigmoid(w)
    zm_ref[...] = zm_gcn * w + zm_mlp * (1.0 - w)
    zs_ref[...] = zs_gcn * r + zs_mlp * (1.0 - r)


def _dense_fuse(x, g2p, mixture_weight, hidden_weight, hidden_bias,
                mean_weight, mean_bias, log_std_weight, log_std_bias):
    mix = mixture_weight.reshape(1, 1)
    row = pl.BlockSpec((BLK, H), lambda i: (i, 0))
    full = lambda shape: pl.BlockSpec(shape, lambda i: tuple(0 for _ in shape))
    return pl.pallas_call(
        _dense_body,
        grid=(N // BLK,),
        in_specs=[
            row,
            full((F, H)), full((H,)),
            full((H, O)), full((O,)),
            full((H, O)), full((O,)),
            pl.BlockSpec((NC, BLK, H), lambda i: (0, i, 0)),
            full((1, 1)),
        ],
        out_specs=[pl.BlockSpec((BLK, O), lambda i: (i, 0))] * 2,
        out_shape=[jax.ShapeDtypeStruct((N, O), jnp.float32)] * 2,
    )(x, hidden_weight, hidden_bias, mean_weight, mean_bias,
      log_std_weight, log_std_bias, g2p, mix)


def kernel(input, edge_index, edge_weight, mixture_weight, hidden_weight,
           hidden_bias, gcn_hidden_weight, gcn_hidden_bias, mean_weight,
           mean_bias, log_std_weight, log_std_bias):
    dst = edge_index[0]
    src = edge_index[1]
    zeros = jnp.zeros((N, H), jnp.float32)
    g1p = _spmm_sc(src, dst, edge_weight, gcn_hidden_weight, zeros)
    hidden_g = _hidden_g(g1p, gcn_hidden_bias)
    g2p = _spmm_sc(src, dst, edge_weight, hidden_g, zeros)
    zm, zs = _dense_fuse(input, g2p, mixture_weight, hidden_weight,
                         hidden_bias, mean_weight, mean_bias,
                         log_std_weight, log_std_bias)
    return (zm, zs)


# speedup vs baseline: 1.1209x; 1.0029x over previous
import functools

import jax
import jax.numpy as jnp
from jax import lax
from jax.experimental import pallas as pl
from jax.experimental.pallas import tpu as pltpu
from jax.experimental.pallas import tpu_sc as plsc

N = 10000
F = 128
H = 128
O = 64
E = 320000

NC = 2
NS = 16
NW = NC * NS
EPW = E // NW
CH = 80
NCHUNK = EPW // CH
RPW = 624

BLK = 400


def _spmm_body(src_hbm, dst_hbm, ew_hbm, x_hbm, zero_hbm, out_hbm,
               src_all, ew_all, dst_all, rows_v0, rows_v1,
               acc_sh, sem0, sem1):
    c = lax.axis_index("c")
    s = lax.axis_index("s")
    wid = s * NC + c
    base = wid * EPW
    rows = (rows_v0, rows_v1)
    sems = (sem0, sem1)

    @pl.when(s == 0)
    def _():
        pltpu.sync_copy(zero_hbm, acc_sh)

    pltpu.sync_copy(src_hbm.at[pl.ds(base, EPW)], src_all)
    pltpu.sync_copy(ew_hbm.at[pl.ds(base, EPW)], ew_all)
    pltpu.sync_copy(dst_hbm.at[pl.ds(base, EPW)], dst_all)
    plsc.subcore_barrier()

    def issue(t, p):
        off = t * CH
        pltpu.async_copy(x_hbm.at[src_all.at[pl.ds(off, CH)]], rows[p],
                         sems[p])

    def wait(p):
        pltpu.make_async_copy(x_hbm.at[src_all.at[pl.ds(0, CH)]], rows[p],
                              sems[p]).wait()

    def process(t, p, guard_next):
        wait(p)
        off = t * CH
        for g in range(CH // 16):
            ew16 = ew_all[pl.ds(off + g * 16, 16)]
            for j in range(16):
                e = g * 16 + j
                w = jnp.full((16,), ew16[j], jnp.float32)
                for k in range(H // 16):
                    rows[p][e, pl.ds(16 * k, 16)] = (
                        rows[p][e, pl.ds(16 * k, 16)] * w)
        pltpu.sync_copy(rows[p], acc_sh.at[dst_all.at[pl.ds(off, CH)]],
                        add=True)
        if guard_next:
            @pl.when(t + 2 < NCHUNK)
            def _():
                issue(t + 2, p)
        else:
            issue(t + 2, p)

    issue(0, 0)
    issue(1, 1)

    def pair(i, carry):
        t = i * 2
        process(t, 0, False)
        process(t + 1, 1, True)
        return carry

    lax.fori_loop(0, (NCHUNK - 1) // 2, pair, 0)
    process(NCHUNK - 1, 0, True)
    plsc.subcore_barrier()
    pltpu.sync_copy(acc_sh.at[pl.ds(s * RPW, RPW)],
                    out_hbm.at[c, pl.ds(s * RPW, RPW)])

    @pl.when(s == NS - 1)
    def _():
        pltpu.sync_copy(acc_sh.at[pl.ds(NS * RPW, N - NS * RPW)],
                        out_hbm.at[c, pl.ds(NS * RPW, N - NS * RPW)])


_spmm_sc = pl.kernel(
    _spmm_body,
    out_type=jax.ShapeDtypeStruct((NC, N, H), jnp.float32),
    mesh=plsc.VectorSubcoreMesh(core_axis_name="c", subcore_axis_name="s"),
    scratch_types=[
        pltpu.VMEM((EPW,), jnp.int32),
        pltpu.VMEM((EPW,), jnp.float32),
        pltpu.VMEM((EPW,), jnp.int32),
        pltpu.VMEM((CH, H), jnp.float32),
        pltpu.VMEM((CH, H), jnp.float32),
        pltpu.VMEM_SHARED((N, H), jnp.float32),
        pltpu.SemaphoreType.DMA,
        pltpu.SemaphoreType.DMA,
    ],
)


def _relu_body(gp_ref, b_ref, out_ref):
    out_ref[...] = jnp.maximum(gp_ref[0] + gp_ref[1] + b_ref[...], 0.0)


def _hidden_g(g1p, gcn_hidden_bias):
    return pl.pallas_call(
        _relu_body,
        grid=(N // BLK,),
        in_specs=[
            pl.BlockSpec((NC, BLK, H), lambda i: (0, i, 0)),
            pl.BlockSpec((H,), lambda i: (0,)),
        ],
        out_specs=pl.BlockSpec((BLK, H), lambda i: (i, 0)),
        out_shape=jax.ShapeDtypeStruct((N, H), jnp.float32),
    )(g1p, gcn_hidden_bias)


def _mlp_body(x_ref, hw_ref, hb_ref, mw_ref, mb_ref, lw_ref, lb_ref,
              zm_ref, zs_ref):
    hm = jnp.maximum(x_ref[...] @ hw_ref[...] + hb_ref[...], 0.0)
    zm_ref[...] = hm @ mw_ref[...] + mb_ref[...]
    zs_ref[...] = hm @ lw_ref[...] + lb_ref[...]


def _mlp(x, hidden_weight, hidden_bias, mean_weight, mean_bias,
         log_std_weight, log_std_bias):
    row = pl.BlockSpec((BLK, H), lambda i: (i, 0))
    full = lambda shape: pl.BlockSpec(shape, lambda i: tuple(0 for _ in shape))
    return pl.pallas_call(
        _mlp_body,
        grid=(N // BLK,),
        in_specs=[
            row,
            full((F, H)), full((H,)),
            full((H, O)), full((O,)),
            full((H, O)), full((O,)),
        ],
        out_specs=[pl.BlockSpec((BLK, O), lambda i: (i, 0))] * 2,
        out_shape=[jax.ShapeDtypeStruct((N, O), jnp.float32)] * 2,
    )(x, hidden_weight, hidden_bias, mean_weight, mean_bias,
      log_std_weight, log_std_bias)


def _mix_body(mlp_m_ref, mlp_s_ref, mw_ref, lw_ref, g2p_ref, mix_ref,
              zm_ref, zs_ref):
    g2 = g2p_ref[0] + g2p_ref[1]
    zm_gcn = g2 @ mw_ref[...]
    zs_gcn = g2 @ lw_ref[...]
    w = mix_ref[0, 0]
    r = jax.nn.sigmoid(w)
    zm_ref[...] = zm_gcn * w + mlp_m_ref[...] * (1.0 - w)
    zs_ref[...] = zs_gcn * r + mlp_s_ref[...] * (1.0 - r)


def _mix(mlp_m, mlp_s, mean_weight, log_std_weight, g2p, mixture_weight):
    mix = mixture_weight.reshape(1, 1)
    rowo = pl.BlockSpec((BLK, O), lambda i: (i, 0))
    full = lambda shape: pl.BlockSpec(shape, lambda i: tuple(0 for _ in shape))
    return pl.pallas_call(
        _mix_body,
        grid=(N // BLK,),
        in_specs=[
            rowo, rowo,
            full((H, O)), full((H, O)),
            pl.BlockSpec((NC, BLK, H), lambda i: (0, i, 0)),
            full((1, 1)),
        ],
        out_specs=[rowo] * 2,
        out_shape=[jax.ShapeDtypeStruct((N, O), jnp.float32)] * 2,
    )(mlp_m, mlp_s, mean_weight, log_std_weight, g2p, mix)


def kernel(input, edge_index, edge_weight, mixture_weight, hidden_weight,
           hidden_bias, gcn_hidden_weight, gcn_hidden_bias, mean_weight,
           mean_bias, log_std_weight, log_std_bias):
    dst = edge_index[0]
    src = edge_index[1]
    zeros = jnp.zeros((N, H), jnp.float32)
    g1p = _spmm_sc(src, dst, edge_weight, gcn_hidden_weight, zeros)
    mlp_m, mlp_s = _mlp(input, hidden_weight, hidden_bias, mean_weight,
                        mean_bias, log_std_weight, log_std_bias)
    hidden_g = _hidden_g(g1p, gcn_hidden_bias)
    g2p = _spmm_sc(src, dst, edge_weight, hidden_g, zeros)
    zm, zs = _mix(mlp_m, mlp_s, mean_weight, log_std_weight, g2p,
                  mixture_weight)
    return (zm, zs)


# speedup vs baseline: 1.1443x; 1.0209x over previous
import functools

import jax
import jax.numpy as jnp
from jax import lax
from jax.experimental import pallas as pl
from jax.experimental.pallas import tpu as pltpu
from jax.experimental.pallas import tpu_sc as plsc

N = 10000
F = 128
H = 128
O = 64
E = 320000

NC = 2
NS = 16
NW = NC * NS
EPW = E // NW
CH = 80
NCHUNK = EPW // CH
RPW = 624

BLK = 400


def _spmm_body(src_hbm, dst_hbm, ew_hbm, x_hbm, out_hbm,
               src_all, ew_all, dst_all, rows_v0, rows_v1,
               acc_sh, sem0, sem1):
    c = lax.axis_index("c")
    s = lax.axis_index("s")
    wid = s * NC + c
    base = wid * EPW
    rows = (rows_v0, rows_v1)
    sems = (sem0, sem1)

    zvec = jnp.zeros((16,), jnp.float32)

    def zrow(i, carry):
        for k in range(H // 16):
            rows_v0[i, pl.ds(16 * k, 16)] = zvec
        return carry

    lax.fori_loop(0, CH, zrow, 0)
    for q in range(RPW // CH):
        pltpu.sync_copy(rows_v0, acc_sh.at[pl.ds(s * RPW + q * CH, CH)])
    rem = RPW - (RPW // CH) * CH
    pltpu.sync_copy(rows_v0.at[pl.ds(0, rem)],
                    acc_sh.at[pl.ds(s * RPW + RPW - rem, rem)])

    @pl.when(s == NS - 1)
    def _():
        pltpu.sync_copy(rows_v0.at[pl.ds(0, N - NS * RPW)],
                        acc_sh.at[pl.ds(NS * RPW, N - NS * RPW)])

    pltpu.sync_copy(src_hbm.at[pl.ds(base, EPW)], src_all)
    pltpu.sync_copy(ew_hbm.at[pl.ds(base, EPW)], ew_all)
    pltpu.sync_copy(dst_hbm.at[pl.ds(base, EPW)], dst_all)
    plsc.subcore_barrier()

    def issue(t, p):
        off = t * CH
        pltpu.async_copy(x_hbm.at[src_all.at[pl.ds(off, CH)]], rows[p],
                         sems[p])

    def wait(p):
        pltpu.make_async_copy(x_hbm.at[src_all.at[pl.ds(0, CH)]], rows[p],
                              sems[p]).wait()

    def process(t, p, guard_next):
        wait(p)
        off = t * CH
        for g in range(CH // 16):
            ew16 = ew_all[pl.ds(off + g * 16, 16)]
            for j in range(16):
                e = g * 16 + j
                w = jnp.full((16,), ew16[j], jnp.float32)
                for k in range(H // 16):
                    rows[p][e, pl.ds(16 * k, 16)] = (
                        rows[p][e, pl.ds(16 * k, 16)] * w)
        pltpu.sync_copy(rows[p], acc_sh.at[dst_all.at[pl.ds(off, CH)]],
                        add=True)
        if guard_next:
            @pl.when(t + 2 < NCHUNK)
            def _():
                issue(t + 2, p)
        else:
            issue(t + 2, p)

    issue(0, 0)
    issue(1, 1)

    def pair(i, carry):
        t = i * 2
        process(t, 0, False)
        process(t + 1, 1, True)
        return carry

    lax.fori_loop(0, (NCHUNK - 1) // 2, pair, 0)
    process(NCHUNK - 1, 0, True)
    plsc.subcore_barrier()
    pltpu.sync_copy(acc_sh.at[pl.ds(s * RPW, RPW)],
                    out_hbm.at[c, pl.ds(s * RPW, RPW)])

    @pl.when(s == NS - 1)
    def _():
        pltpu.sync_copy(acc_sh.at[pl.ds(NS * RPW, N - NS * RPW)],
                        out_hbm.at[c, pl.ds(NS * RPW, N - NS * RPW)])


_spmm_sc = pl.kernel(
    _spmm_body,
    out_type=jax.ShapeDtypeStruct((NC, N, H), jnp.float32),
    mesh=plsc.VectorSubcoreMesh(core_axis_name="c", subcore_axis_name="s"),
    scratch_types=[
        pltpu.VMEM((EPW,), jnp.int32),
        pltpu.VMEM((EPW,), jnp.float32),
        pltpu.VMEM((EPW,), jnp.int32),
        pltpu.VMEM((CH, H), jnp.float32),
        pltpu.VMEM((CH, H), jnp.float32),
        pltpu.VMEM_SHARED((N, H), jnp.float32),
        pltpu.SemaphoreType.DMA,
        pltpu.SemaphoreType.DMA,
    ],
)


def _relu_body(gp_ref, b_ref, out_ref):
    out_ref[...] = jnp.maximum(gp_ref[0] + gp_ref[1] + b_ref[...], 0.0)


def _hidden_g(g1p, gcn_hidden_bias):
    return pl.pallas_call(
        _relu_body,
        grid=(N // BLK,),
        in_specs=[
            pl.BlockSpec((NC, BLK, H), lambda i: (0, i, 0)),
            pl.BlockSpec((H,), lambda i: (0,)),
        ],
        out_specs=pl.BlockSpec((BLK, H), lambda i: (i, 0)),
        out_shape=jax.ShapeDtypeStruct((N, H), jnp.float32),
    )(g1p, gcn_hidden_bias)


def _mlp_body(x_ref, hw_ref, hb_ref, mw_ref, mb_ref, lw_ref, lb_ref,
              zm_ref, zs_ref):
    hm = jnp.maximum(x_ref[...] @ hw_ref[...] + hb_ref[...], 0.0)
    zm_ref[...] = hm @ mw_ref[...] + mb_ref[...]
    zs_ref[...] = hm @ lw_ref[...] + lb_ref[...]


def _mlp(x, hidden_weight, hidden_bias, mean_weight, mean_bias,
         log_std_weight, log_std_bias):
    row = pl.BlockSpec((BLK, H), lambda i: (i, 0))
    full = lambda shape: pl.BlockSpec(shape, lambda i: tuple(0 for _ in shape))
    return pl.pallas_call(
        _mlp_body,
        grid=(N // BLK,),
        in_specs=[
            row,
            full((F, H)), full((H,)),
            full((H, O)), full((O,)),
            full((H, O)), full((O,)),
        ],
        out_specs=[pl.BlockSpec((BLK, O), lambda i: (i, 0))] * 2,
        out_shape=[jax.ShapeDtypeStruct((N, O), jnp.float32)] * 2,
    )(x, hidden_weight, hidden_bias, mean_weight, mean_bias,
      log_std_weight, log_std_bias)


def _mix_body(mlp_m_ref, mlp_s_ref, mw_ref, lw_ref, g2p_ref, mix_ref,
              zm_ref, zs_ref):
    g2 = g2p_ref[0] + g2p_ref[1]
    zm_gcn = g2 @ mw_ref[...]
    zs_gcn = g2 @ lw_ref[...]
    w = mix_ref[0, 0]
    r = jax.nn.sigmoid(w)
    zm_ref[...] = zm_gcn * w + mlp_m_ref[...] * (1.0 - w)
    zs_ref[...] = zs_gcn * r + mlp_s_ref[...] * (1.0 - r)


def _mix(mlp_m, mlp_s, mean_weight, log_std_weight, g2p, mixture_weight):
    mix = mixture_weight.reshape(1, 1)
    rowo = pl.BlockSpec((BLK, O), lambda i: (i, 0))
    full = lambda shape: pl.BlockSpec(shape, lambda i: tuple(0 for _ in shape))
    return pl.pallas_call(
        _mix_body,
        grid=(N // BLK,),
        in_specs=[
            rowo, rowo,
            full((H, O)), full((H, O)),
            pl.BlockSpec((NC, BLK, H), lambda i: (0, i, 0)),
            full((1, 1)),
        ],
        out_specs=[rowo] * 2,
        out_shape=[jax.ShapeDtypeStruct((N, O), jnp.float32)] * 2,
    )(mlp_m, mlp_s, mean_weight, log_std_weight, g2p, mix)


def kernel(input, edge_index, edge_weight, mixture_weight, hidden_weight,
           hidden_bias, gcn_hidden_weight, gcn_hidden_bias, mean_weight,
           mean_bias, log_std_weight, log_std_bias):
    dst = edge_index[0]
    src = edge_index[1]
    g1p = _spmm_sc(src, dst, edge_weight, gcn_hidden_weight)
    mlp_m, mlp_s = _mlp(input, hidden_weight, hidden_bias, mean_weight,
                        mean_bias, log_std_weight, log_std_bias)
    hidden_g = _hidden_g(g1p, gcn_hidden_bias)
    g2p = _spmm_sc(src, dst, edge_weight, hidden_g)
    zm, zs = _mix(mlp_m, mlp_s, mean_weight, log_std_weight, g2p,
                  mixture_weight)
    return (zm, zs)
